# baseline (speedup 1.0000x reference)
import functools
import math

import jax
import jax.numpy as jnp
import numpy as np
from jax import lax
from jax.experimental import pallas as pl
from jax.experimental.pallas import tpu as pltpu
from jax.experimental.pallas import tpu_sc as plsc

_SC_NC, _SC_NS = 2, 16
_SC_NW = _SC_NC * _SC_NS

DIM = 768
HEADS = 12
HEAD_DIM = 64
K_NEIGH = 64
SCALE = 1.0 / math.sqrt(HEAD_DIM)
NEG = -1e30
QB = 256
WIN = 384

_CONTRACT_LAST = (((1,), (1,)), ((), ()))


@functools.lru_cache(maxsize=None)
def _route_constants(seq_len: int, k: int, depth: int = 8):
    pos = np.arange(seq_len)
    x = pos.astype(np.float32) / np.float32(max(1, seq_len - 1))
    x = np.clip(x, np.float32(1e-06), np.float32(1.0 - 1e-06)).astype(np.float32)
    val = np.zeros_like(x)
    factor = 0.5
    for _ in range(depth):
        x_scaled = x * np.float32(3.0)
        digit = x_scaled.astype(np.int32)
        x_frac = (x_scaled - digit.astype(np.float32)).astype(np.float32)
        val = (val + (digit == 2).astype(np.float32) * np.float32(factor)).astype(np.float32)
        x = x_frac
        factor *= 0.5
    val = np.clip(val, 0.0, 1.0).astype(np.float32)
    dist = np.abs(val[:, None] - val[None, :])
    routes = np.argsort(dist, axis=1, kind="stable")[:, :k]

    perm = np.argsort(val, kind="stable")
    rank = np.empty(seq_len, dtype=np.int64)
    rank[perm] = np.arange(seq_len)

    nbr_ranks = rank[routes]
    nbr_sorted = nbr_ranks[perm]
    lo = nbr_sorted.min(axis=1)
    hi = nbr_sorted.max(axis=1)

    n_blocks = seq_len // QB
    ws = np.zeros(n_blocks, dtype=np.int32)
    bias = np.full((seq_len, WIN), NEG, dtype=np.float32)
    for b in range(n_blocks):
        r0, r1 = b * QB, (b + 1) * QB
        start = (lo[r0:r1].min() // 16) * 16
        start = min(int(start), seq_len - WIN)
        assert hi[r0:r1].max() < start + WIN
        ws[b] = start
        for r in range(r0, r1):
            bias[r, nbr_sorted[r] - start] = 0.0
    return perm.astype(np.int32), rank.astype(np.int32), ws, bias


def _sc_permute_rows(table, idx):
    n, d = table.shape
    rows_per_w = n // _SC_NW
    mesh = plsc.VectorSubcoreMesh(core_axis_name="c", subcore_axis_name="s")

    @functools.partial(
        pl.kernel, mesh=mesh,
        out_type=jax.ShapeDtypeStruct((n, d), table.dtype),
        scratch_types=[
            pltpu.VMEM((rows_per_w,), jnp.int32),
            pltpu.VMEM((rows_per_w, d), table.dtype),
            pltpu.SemaphoreType.DMA,
        ],
    )
    def k(table_hbm, idx_hbm, out_hbm, idx_v, rows_v, sem):
        wid = lax.axis_index("s") * _SC_NC + lax.axis_index("c")
        base = wid * rows_per_w
        pltpu.sync_copy(idx_hbm.at[pl.ds(base, rows_per_w)], idx_v)
        pltpu.async_copy(table_hbm.at[idx_v], rows_v, sem).wait()
        pltpu.sync_copy(rows_v, out_hbm.at[pl.ds(base, rows_per_w)])

    return k(table, idx)


def _fused_kernel(ws_ref, x_ref, w_ref, b_ref, wo_ref, bo_ref, bias_ref,
                  o_ref, k_scr, v_scr, oh_scr):
    b = pl.program_id(0)

    @pl.when(b == 0)
    def _prime_kv():
        x = x_ref[...]
        k_scr[...] = jax.lax.dot_general(
            x, w_ref[DIM:2 * DIM], _CONTRACT_LAST,
            preferred_element_type=jnp.float32) + b_ref[0, DIM:2 * DIM]
        v_scr[...] = jax.lax.dot_general(
            x, w_ref[2 * DIM:], _CONTRACT_LAST,
            preferred_element_type=jnp.float32) + b_ref[0, 2 * DIM:]

    ws = pl.multiple_of(ws_ref[b], 16)
    kw = k_scr[pl.ds(ws, WIN), :]
    vw = v_scr[pl.ds(ws, WIN), :]
    bias_b = bias_ref[...]
    qb_all = (
        jax.lax.dot_general(x_ref[pl.ds(b * QB, QB)], w_ref[:DIM],
                            _CONTRACT_LAST,
                            preferred_element_type=jnp.float32)
        + b_ref[0, :DIM]
    ) * SCALE
    for h in range(HEADS):
        c0, c1 = h * HEAD_DIM, (h + 1) * HEAD_DIM
        qb = qb_all[:, c0:c1]
        s = (
            jnp.dot(qb, kw[:, c0:c1].T, preferred_element_type=jnp.float32)
            + bias_b
        )
        m = jnp.max(s, axis=-1, keepdims=True)
        e = jnp.exp(s - m)
        o_hb = jnp.dot(e, vw[:, c0:c1], preferred_element_type=jnp.float32)
        o_hb = o_hb / jnp.sum(e, axis=-1, keepdims=True)
        oh_scr[:, c0:c1] = o_hb
    o_ref[...] = (
        jax.lax.dot_general(oh_scr[...], wo_ref[...], _CONTRACT_LAST,
                            preferred_element_type=jnp.float32)
        + bo_ref[...]
    )


def kernel(x, W_qkv, b_qkv, W_out, b_out):
    B, S, D = x.shape
    H, hd = HEADS, HEAD_DIM
    perm_np, rank_np, ws_np, bias_np = _route_constants(S, K_NEIGH)
    perm = jnp.asarray(perm_np)
    invperm = jnp.asarray(rank_np)
    ws = jnp.asarray(ws_np)
    bias = jnp.asarray(bias_np)

    x_perm = _sc_permute_rows(x.reshape(S, D), perm)

    out = pl.pallas_call(
        _fused_kernel,
        grid=(S // QB,),
        in_specs=[
            pl.BlockSpec(memory_space=pltpu.SMEM),
            pl.BlockSpec((S, D), lambda b: (0, 0)),
            pl.BlockSpec((3 * D, D), lambda b: (0, 0)),
            pl.BlockSpec((1, 3 * D), lambda b: (0, 0)),
            pl.BlockSpec((D, D), lambda b: (0, 0)),
            pl.BlockSpec((1, D), lambda b: (0, 0)),
            pl.BlockSpec((QB, WIN), lambda b: (b, 0)),
        ],
        out_specs=pl.BlockSpec((QB, D), lambda b: (b, 0)),
        out_shape=jax.ShapeDtypeStruct((S, D), jnp.float32),
        scratch_shapes=[
            pltpu.VMEM((S, D), jnp.float32),
            pltpu.VMEM((S, D), jnp.float32),
            pltpu.VMEM((QB, D), jnp.float32),
        ],
    )(ws, x_perm, W_qkv, b_qkv.reshape(1, 3 * D), W_out, b_out.reshape(1, D),
      bias)

    return _sc_permute_rows(out, invperm).reshape(B, S, D)
